# Initial kernel scaffold; baseline (speedup 1.0000x reference)
import functools

import jax
import jax.numpy as jnp
from jax import lax
from jax.experimental import pallas as pl
from jax.experimental.pallas import tpu as pltpu
from jax.experimental.pallas import tpu_sc as plsc

N_NODES = 10000
N_EDGES = 320000
D = 128

NC = 2
NS = 16
NW = NC * NS

SEG_PER_W = 320
N_SEG_PAD = NW * SEG_PER_W
PTR_SLICE = 328
PTR_PAD = (NW - 1) * SEG_PER_W + PTR_SLICE
CH = 128
NLANES = D // 16


def _seg_max_sc(x_mod, ptr32):
    mesh = plsc.VectorSubcoreMesh(
        core_axis_name="c", subcore_axis_name="s", num_cores=NC, num_subcores=NS
    )

    @functools.partial(
        pl.kernel,
        out_type=[
            jax.ShapeDtypeStruct((N_SEG_PAD, D), jnp.float32),
            jax.ShapeDtypeStruct((N_SEG_PAD,), jnp.int32),
        ],
        mesh=mesh,
        scratch_types=[
            pltpu.VMEM((PTR_SLICE,), jnp.int32),
            pltpu.VMEM((SEG_PER_W, D), jnp.float32),
            pltpu.VMEM((SEG_PER_W,), jnp.int32),
            pltpu.VMEM((2, CH, D), jnp.float32),
            pltpu.SemaphoreType.DMA,
            pltpu.SemaphoreType.DMA,
        ],
    )
    def seg_max(x_mod_hbm, ptr_hbm, pool_hbm, seen_hbm,
                ptr_v, acc_v, seen_v, buf_v, sem0, sem1):
        wid = lax.axis_index("s") * NC + lax.axis_index("c")
        s0 = wid * SEG_PER_W

        pltpu.sync_copy(ptr_hbm.at[pl.ds(s0, PTR_SLICE)], ptr_v)
        e_lo = ptr_v[0]
        e_hi = ptr_v[SEG_PER_W]

        n_edges = e_hi - e_lo
        nchunks = lax.div(n_edges + (CH - 1), CH)

        @pl.when(nchunks > 0)
        def _():
            st = jnp.minimum(e_lo, N_EDGES - CH)
            pltpu.make_async_copy(
                x_mod_hbm.at[pl.ds(st, CH)], buf_v.at[0], sem0
            ).start()

        zeros16 = jnp.zeros((16,), jnp.float32)

        def zrow(r, _):
            for j in range(NLANES):
                acc_v[r, pl.ds(16 * j, 16)] = zeros16
            return 0

        lax.fori_loop(0, SEG_PER_W, zrow, 0)

        iota = jnp.arange(16, dtype=jnp.int32)
        for k in range(SEG_PER_W // 16):
            a = ptr_v[pl.ds(16 * k, 16)]
            b = plsc.load_gather(ptr_v, [iota + (16 * k + 1)])
            seen_v[pl.ds(16 * k, 16)] = jnp.where(b > a, 1, 0).astype(jnp.int32)

        minus_inf = jnp.full((16,), -jnp.inf, jnp.float32)

        def process_chunk(c, b, carry):
            st_nom = e_lo + c * CH
            st_dma = jnp.minimum(st_nom, N_EDGES - CH)
            delta = st_nom - st_dma
            count = jnp.minimum(CH, e_hi - st_nom)

            def row_body(r, carry):
                cur, has, *acc = carry
                e = st_nom + r
                nxt = ptr_v[cur + 1]

                def flush(ops):
                    fcur, fhas, *facc = ops

                    @pl.when(fhas != 0)
                    def _():
                        for j in range(NLANES):
                            acc_v[fcur, pl.ds(16 * j, 16)] = facc[j]

                    ncur = lax.while_loop(
                        lambda cc: ptr_v[cc + 1] <= e, lambda cc: cc + 1, fcur
                    )
                    return (ncur, jnp.int32(0)) + tuple([minus_inf] * NLANES)

                cur, has, *acc = lax.cond(
                    e >= nxt, flush, lambda ops: ops, (cur, has, *acc)
                )
                bi = r + delta
                newacc = []
                for j in range(NLANES):
                    row = buf_v[b, bi, pl.ds(16 * j, 16)]
                    newacc.append(jnp.maximum(acc[j], row))
                return (cur, jnp.int32(1)) + tuple(newacc)

            return lax.fori_loop(0, count, row_body, carry)

        init = (jnp.int32(0), jnp.int32(0)) + tuple([minus_inf] * NLANES)
        npairs = lax.div(nchunks + 1, 2)

        def pair_body(p, carry):
            c0 = 2 * p
            c1 = c0 + 1

            @pl.when(c1 < nchunks)
            def _():
                st = jnp.minimum(e_lo + c1 * CH, N_EDGES - CH)
                pltpu.make_async_copy(
                    x_mod_hbm.at[pl.ds(st, CH)], buf_v.at[1], sem1
                ).start()

            pltpu.make_async_copy(
                x_mod_hbm.at[pl.ds(0, CH)], buf_v.at[0], sem0
            ).wait()
            carry = process_chunk(c0, 0, carry)

            @pl.when(c0 + 2 < nchunks)
            def _():
                st = jnp.minimum(e_lo + (c0 + 2) * CH, N_EDGES - CH)
                pltpu.make_async_copy(
                    x_mod_hbm.at[pl.ds(st, CH)], buf_v.at[0], sem0
                ).start()

            def do_c1(cr):
                pltpu.make_async_copy(
                    x_mod_hbm.at[pl.ds(0, CH)], buf_v.at[1], sem1
                ).wait()
                return process_chunk(c1, 1, cr)

            return lax.cond(c1 < nchunks, do_c1, lambda cr: cr, carry)

        cur, has, *acc = lax.fori_loop(0, npairs, pair_body, init)

        @pl.when(has != 0)
        def _():
            for j in range(NLANES):
                acc_v[cur, pl.ds(16 * j, 16)] = acc[j]

        pltpu.sync_copy(acc_v, pool_hbm.at[pl.ds(s0, SEG_PER_W)])
        pltpu.sync_copy(seen_v, seen_hbm.at[pl.ds(s0, SEG_PER_W)])

    return seg_max(x_mod, ptr32)


def kernel(x_main, x_mod, csr_idx):
    del x_main
    n_seg = csr_idx.shape[0] - 1
    ptr32 = jnp.concatenate(
        [
            csr_idx.astype(jnp.int32),
            jnp.full((PTR_PAD - n_seg - 1,), N_EDGES, jnp.int32),
        ]
    )
    pool_pad, seen_pad = _seg_max_sc(x_mod, ptr32)
    return (pool_pad[:n_seg], seen_pad[:n_seg].astype(bool))


# speedup vs baseline: 76.3881x; 76.3881x over previous
import functools

import jax
import jax.numpy as jnp
from jax import lax
from jax.experimental import pallas as pl
from jax.experimental.pallas import tpu as pltpu
from jax.experimental.pallas import tpu_sc as plsc

N_NODES = 10000
N_EDGES = 320000
D = 128

NC = 2
NS = 16
NW = NC * NS

SEG_PER_W = 320
N_SEG_PAD = NW * SEG_PER_W
PTR_SLICE = 344
PTR_PAD = (NW - 1) * SEG_PER_W + PTR_SLICE
CH = 128
NLANES = D // 16


def _seg_max_sc(x_mod_flat, ptr32):
    mesh = plsc.VectorSubcoreMesh(
        core_axis_name="c", subcore_axis_name="s", num_cores=NC, num_subcores=NS
    )

    @functools.partial(
        pl.kernel,
        out_type=[
            jax.ShapeDtypeStruct((N_SEG_PAD * D,), jnp.float32),
            jax.ShapeDtypeStruct((N_SEG_PAD,), jnp.int32),
        ],
        mesh=mesh,
        scratch_types=[
            pltpu.VMEM((PTR_SLICE,), jnp.int32),
            pltpu.VMEM((SEG_PER_W * D,), jnp.float32),
            pltpu.VMEM((SEG_PER_W,), jnp.int32),
            pltpu.VMEM((2 * CH * D,), jnp.float32),
            pltpu.SemaphoreType.DMA,
            pltpu.SemaphoreType.DMA,
        ],
    )
    def seg_max(x_mod_hbm, ptr_hbm, pool_hbm, seen_hbm,
                ptr_v, acc_v, seen_v, buf_v, sem0, sem1):
        wid = lax.axis_index("s") * NC + lax.axis_index("c")
        s0 = wid * SEG_PER_W

        pltpu.sync_copy(ptr_hbm.at[pl.ds(s0, PTR_SLICE)], ptr_v)

        def ptr_at(i):
            return ptr_v[pl.ds(i, 16)][0]

        e_lo = ptr_at(0)
        e_hi = ptr_at(SEG_PER_W)

        base = (e_lo // 8) * 8
        nchunks = jnp.where(
            e_hi > e_lo, lax.div(e_hi - base + (CH - 1), CH), 0
        )

        def chunk_start(c):
            st = jnp.minimum(base + c * CH, N_EDGES - CH)
            return pl.multiple_of(st, 8)

        def start_dma(c, b, sem):
            pltpu.make_async_copy(
                x_mod_hbm.at[pl.ds(chunk_start(c) * D, CH * D)],
                buf_v.at[pl.ds(b * CH * D, CH * D)],
                sem,
            ).start()

        def wait_dma(b, sem):
            pltpu.make_async_copy(
                x_mod_hbm.at[pl.ds(0, CH * D)],
                buf_v.at[pl.ds(b * CH * D, CH * D)],
                sem,
            ).wait()

        @pl.when(nchunks > 0)
        def _():
            start_dma(0, 0, sem0)

        @pl.when(nchunks > 1)
        def _():
            start_dma(1, 1, sem1)

        zeros16 = jnp.zeros((16,), jnp.float32)

        def zrow(r, _):
            for j in range(NLANES):
                acc_v[pl.ds(r * D + 16 * j, 16)] = zeros16
            return 0

        lax.fori_loop(0, SEG_PER_W, zrow, 0)

        for k in range(SEG_PER_W // 16):
            a = ptr_v[pl.ds(16 * k, 16)]
            b = ptr_v[pl.ds(16 * k + 1, 16)]
            seen_v[pl.ds(16 * k, 16)] = jnp.where(b > a, 1, 0).astype(jnp.int32)

        minus_inf = jnp.full((16,), -jnp.inf, jnp.float32)

        def row_body(e, ops):
            res, *acc = ops
            c = lax.div(e - base, CH)

            def do_cross(_):
                @pl.when(lax.rem(c, 2) == 0)
                def _():
                    wait_dma(0, sem0)

                @pl.when(lax.rem(c, 2) == 1)
                def _():
                    wait_dma(1, sem1)

                nc = c + 1

                @pl.when((c > 0) & (nc < nchunks))
                def _():
                    @pl.when(lax.rem(nc, 2) == 0)
                    def _():
                        start_dma(nc, 0, sem0)

                    @pl.when(lax.rem(nc, 2) == 1)
                    def _():
                        start_dma(nc, 1, sem1)

                return c

            res = lax.cond(c > res, do_cross, lambda _: res, 0)

            st_dma = jnp.minimum(base + c * CH, N_EDGES - CH)
            bi = (lax.rem(c, 2) * CH + (e - st_dma)) * D
            newacc = []
            for j in range(NLANES):
                row = buf_v[pl.ds(bi + 16 * j, 16)]
                newacc.append(jnp.maximum(acc[j], row))
            return (res,) + tuple(newacc)

        def seg_body(s, res):
            start = ptr_at(s)
            end = ptr_at(s + 1)
            init = (res,) + tuple([minus_inf] * NLANES)
            res, *acc = lax.fori_loop(start, end, row_body, init)

            @pl.when(end > start)
            def _():
                for j in range(NLANES):
                    acc_v[pl.ds(s * D + 16 * j, 16)] = acc[j]

            return res

        lax.fori_loop(0, SEG_PER_W, seg_body, jnp.int32(-1))

        pltpu.sync_copy(acc_v, pool_hbm.at[pl.ds(s0 * D, SEG_PER_W * D)])
        pltpu.sync_copy(seen_v, seen_hbm.at[pl.ds(s0, SEG_PER_W)])

    return seg_max(x_mod_flat, ptr32)


def kernel(x_main, x_mod, csr_idx):
    del x_main
    n_seg = csr_idx.shape[0] - 1
    ptr32 = jnp.concatenate(
        [
            csr_idx.astype(jnp.int32),
            jnp.full((PTR_PAD - n_seg - 1,), N_EDGES, jnp.int32),
        ]
    )
    pool_pad, seen_pad = _seg_max_sc(x_mod.reshape(-1), ptr32)
    pool = pool_pad.reshape(N_SEG_PAD, D)[:n_seg]
    return (pool, seen_pad[:n_seg].astype(bool))


# speedup vs baseline: 168.3322x; 2.2036x over previous
import functools

import jax
import jax.numpy as jnp
from jax import lax
from jax.experimental import pallas as pl
from jax.experimental.pallas import tpu as pltpu
from jax.experimental.pallas import tpu_sc as plsc

N_NODES = 10000
N_EDGES = 320000
D = 128

NC = 2
NS = 16
NW = NC * NS

SEG_PER_W = 320
N_SEG_PAD = NW * SEG_PER_W
PTR_SLICE = 344
PTR_PAD = (NW - 1) * SEG_PER_W + PTR_SLICE
CH = 128
NLANES = D // 16


def _seg_max_sc(x_mod_flat, ptr32):
    mesh = plsc.VectorSubcoreMesh(
        core_axis_name="c", subcore_axis_name="s", num_cores=NC, num_subcores=NS
    )

    @functools.partial(
        pl.kernel,
        out_type=[
            jax.ShapeDtypeStruct((N_SEG_PAD * D,), jnp.float32),
            jax.ShapeDtypeStruct((N_SEG_PAD,), jnp.int32),
        ],
        mesh=mesh,
        scratch_types=[
            pltpu.VMEM((PTR_SLICE,), jnp.int32),
            pltpu.VMEM((SEG_PER_W * D,), jnp.float32),
            pltpu.VMEM((SEG_PER_W,), jnp.int32),
            pltpu.VMEM((2 * CH * D,), jnp.float32),
            pltpu.SemaphoreType.DMA,
            pltpu.SemaphoreType.DMA,
        ],
    )
    def seg_max(x_mod_hbm, ptr_hbm, pool_hbm, seen_hbm,
                ptr_v, acc_v, seen_v, buf_v, sem0, sem1):
        wid = lax.axis_index("s") * NC + lax.axis_index("c")
        s0 = wid * SEG_PER_W

        pltpu.sync_copy(ptr_hbm.at[pl.ds(s0, PTR_SLICE)], ptr_v)

        def ptr_at(i):
            return ptr_v[pl.ds(i, 16)][0]

        e_lo = ptr_at(0)
        e_hi = ptr_at(SEG_PER_W)

        base = (e_lo // 8) * 8
        nchunks = jnp.where(
            e_hi > e_lo, lax.div(e_hi - base + (CH - 1), CH), 0
        )

        def chunk_start(c):
            st = jnp.minimum(base + c * CH, N_EDGES - CH)
            return pl.multiple_of(st, 8)

        def start_dma(c, b, sem):
            pltpu.make_async_copy(
                x_mod_hbm.at[pl.ds(chunk_start(c) * D, CH * D)],
                buf_v.at[pl.ds(b * CH * D, CH * D)],
                sem,
            ).start()

        def wait_dma(b, sem):
            pltpu.make_async_copy(
                x_mod_hbm.at[pl.ds(0, CH * D)],
                buf_v.at[pl.ds(b * CH * D, CH * D)],
                sem,
            ).wait()

        @pl.when(nchunks > 0)
        def _():
            start_dma(0, 0, sem0)

        @pl.when(nchunks > 1)
        def _():
            start_dma(1, 1, sem1)

        zeros16 = jnp.zeros((16,), jnp.float32)

        def zrow(r, _):
            for j in range(NLANES):
                acc_v[pl.ds(r * D + 16 * j, 16)] = zeros16
            return 0

        lax.fori_loop(0, SEG_PER_W, zrow, 0)

        for k in range(SEG_PER_W // 16):
            a = ptr_v[pl.ds(16 * k, 16)]
            b = ptr_v[pl.ds(16 * k + 1, 16)]
            seen_v[pl.ds(16 * k, 16)] = jnp.where(b > a, 1, 0).astype(jnp.int32)

        minus_inf = jnp.full((16,), -jnp.inf, jnp.float32)

        def seg_body(s, ops):
            res, start = ops
            end = ptr_at(s + 1)

            c_first = lax.div(start - base, CH)
            c_last = jnp.where(
                end > start, lax.div(end - 1 - base, CH), c_first - 1
            )

            def chunk_body(c, cops):
                res, *acc = cops

                def do_cross(_):
                    @pl.when(lax.rem(c, 2) == 0)
                    def _():
                        wait_dma(0, sem0)

                    @pl.when(lax.rem(c, 2) == 1)
                    def _():
                        wait_dma(1, sem1)

                    nc = c + 1

                    @pl.when((c > 0) & (nc < nchunks))
                    def _():
                        @pl.when(lax.rem(nc, 2) == 0)
                        def _():
                            start_dma(nc, 0, sem0)

                        @pl.when(lax.rem(nc, 2) == 1)
                        def _():
                            start_dma(nc, 1, sem1)

                    return c

                res = lax.cond(c > res, do_cross, lambda _: res, 0)

                st_dma = jnp.minimum(base + c * CH, N_EDGES - CH)
                off = (lax.rem(c, 2) * CH - st_dma) * D
                lo = jnp.maximum(start, base + c * CH)
                hi = jnp.minimum(end, base + (c + 1) * CH)

                def row_body(e, acc):
                    bi = off + e * D
                    return tuple(
                        jnp.maximum(acc[j], buf_v[pl.ds(bi + 16 * j, 16)])
                        for j in range(NLANES)
                    )

                acc = lax.fori_loop(lo, hi, row_body, tuple(acc))
                return (res,) + acc

            init = (res,) + tuple([minus_inf] * NLANES)
            res, *acc = lax.fori_loop(c_first, c_last + 1, chunk_body, init)

            @pl.when(end > start)
            def _():
                for j in range(NLANES):
                    acc_v[pl.ds(s * D + 16 * j, 16)] = acc[j]

            return (res, end)

        lax.fori_loop(0, SEG_PER_W, seg_body, (jnp.int32(-1), e_lo))

        pltpu.sync_copy(acc_v, pool_hbm.at[pl.ds(s0 * D, SEG_PER_W * D)])
        pltpu.sync_copy(seen_v, seen_hbm.at[pl.ds(s0, SEG_PER_W)])

    return seg_max(x_mod_flat, ptr32)


def kernel(x_main, x_mod, csr_idx):
    del x_main
    n_seg = csr_idx.shape[0] - 1
    ptr32 = jnp.concatenate(
        [
            csr_idx.astype(jnp.int32),
            jnp.full((PTR_PAD - n_seg - 1,), N_EDGES, jnp.int32),
        ]
    )
    pool_pad, seen_pad = _seg_max_sc(x_mod.reshape(-1), ptr32)
    pool = pool_pad.reshape(N_SEG_PAD, D)[:n_seg]
    return (pool, seen_pad[:n_seg].astype(bool))


# speedup vs baseline: 206.4357x; 1.2264x over previous
import functools

import jax
import jax.numpy as jnp
from jax import lax
from jax.experimental import pallas as pl
from jax.experimental.pallas import tpu as pltpu
from jax.experimental.pallas import tpu_sc as plsc

N_NODES = 10000
N_EDGES = 320000
D = 128

NC = 2
NS = 16
NW = NC * NS

SEG_PER_W = 320
N_SEG_PAD = NW * SEG_PER_W
PTR_SLICE = 344
PTR_PAD = (NW - 1) * SEG_PER_W + PTR_SLICE
CH = 256
NLANES = D // 16


def _seg_max_sc(x_mod_flat, ptr32):
    mesh = plsc.VectorSubcoreMesh(
        core_axis_name="c", subcore_axis_name="s", num_cores=NC, num_subcores=NS
    )

    @functools.partial(
        pl.kernel,
        out_type=[
            jax.ShapeDtypeStruct((N_SEG_PAD * D,), jnp.float32),
            jax.ShapeDtypeStruct((N_SEG_PAD,), jnp.int32),
        ],
        mesh=mesh,
        scratch_types=[
            pltpu.VMEM((PTR_SLICE,), jnp.int32),
            pltpu.VMEM((SEG_PER_W * D,), jnp.float32),
            pltpu.VMEM((SEG_PER_W,), jnp.int32),
            pltpu.VMEM((2 * CH * D,), jnp.float32),
            pltpu.SemaphoreType.DMA,
            pltpu.SemaphoreType.DMA,
        ],
    )
    def seg_max(x_mod_hbm, ptr_hbm, pool_hbm, seen_hbm,
                ptr_v, acc_v, seen_v, buf_v, sem0, sem1):
        wid = lax.axis_index("s") * NC + lax.axis_index("c")
        s0 = wid * SEG_PER_W

        pltpu.sync_copy(ptr_hbm.at[pl.ds(s0, PTR_SLICE)], ptr_v)

        def ptr_at(i):
            return ptr_v[pl.ds(i, 16)][0]

        e_lo = ptr_at(0)
        e_hi = ptr_at(SEG_PER_W)

        base = (e_lo // 8) * 8
        nchunks = jnp.where(
            e_hi > e_lo, lax.div(e_hi - base + (CH - 1), CH), 0
        )

        def chunk_start(c):
            st = jnp.minimum(base + c * CH, N_EDGES - CH)
            return pl.multiple_of(st, 8)

        def start_dma(c, b, sem):
            pltpu.make_async_copy(
                x_mod_hbm.at[pl.ds(chunk_start(c) * D, CH * D)],
                buf_v.at[pl.ds(b * CH * D, CH * D)],
                sem,
            ).start()

        def wait_dma(b, sem):
            pltpu.make_async_copy(
                x_mod_hbm.at[pl.ds(0, CH * D)],
                buf_v.at[pl.ds(b * CH * D, CH * D)],
                sem,
            ).wait()

        @pl.when(nchunks > 0)
        def _():
            start_dma(0, 0, sem0)

        @pl.when(nchunks > 1)
        def _():
            start_dma(1, 1, sem1)

        zeros16 = jnp.zeros((16,), jnp.float32)

        def zrow(r, _):
            for j in range(NLANES):
                acc_v[pl.ds(r * D + 16 * j, 16)] = zeros16
            return 0

        lax.fori_loop(0, SEG_PER_W, zrow, 0)

        for k in range(SEG_PER_W // 16):
            a = ptr_v[pl.ds(16 * k, 16)]
            b = ptr_v[pl.ds(16 * k + 1, 16)]
            seen_v[pl.ds(16 * k, 16)] = jnp.where(b > a, 1, 0).astype(jnp.int32)

        minus_inf = jnp.full((16,), -jnp.inf, jnp.float32)

        def seg_body(s, ops):
            res, start = ops
            end = ptr_at(s + 1)

            c_first = lax.div(start - base, CH)
            c_last = jnp.where(
                end > start, lax.div(end - 1 - base, CH), c_first - 1
            )

            def chunk_body(c, cops):
                res, *acc = cops

                def do_cross(_):
                    @pl.when(lax.rem(c, 2) == 0)
                    def _():
                        wait_dma(0, sem0)

                    @pl.when(lax.rem(c, 2) == 1)
                    def _():
                        wait_dma(1, sem1)

                    nc = c + 1

                    @pl.when((c > 0) & (nc < nchunks))
                    def _():
                        @pl.when(lax.rem(nc, 2) == 0)
                        def _():
                            start_dma(nc, 0, sem0)

                        @pl.when(lax.rem(nc, 2) == 1)
                        def _():
                            start_dma(nc, 1, sem1)

                    return c

                res = lax.cond(c > res, do_cross, lambda _: res, 0)

                st_dma = jnp.minimum(base + c * CH, N_EDGES - CH)
                off = (lax.rem(c, 2) * CH - st_dma) * D
                lo = jnp.maximum(start, base + c * CH)
                hi = jnp.minimum(end, base + (c + 1) * CH)

                def row_body(e, acc):
                    bi = off + e * D
                    return tuple(
                        jnp.maximum(acc[j], buf_v[pl.ds(bi + 16 * j, 16)])
                        for j in range(NLANES)
                    )

                acc = lax.fori_loop(lo, hi, row_body, tuple(acc))
                return (res,) + acc

            init = (res,) + tuple([minus_inf] * NLANES)
            res, *acc = lax.fori_loop(c_first, c_last + 1, chunk_body, init)

            @pl.when(end > start)
            def _():
                for j in range(NLANES):
                    acc_v[pl.ds(s * D + 16 * j, 16)] = acc[j]

            return (res, end)

        lax.fori_loop(0, SEG_PER_W, seg_body, (jnp.int32(-1), e_lo))

        pltpu.sync_copy(acc_v, pool_hbm.at[pl.ds(s0 * D, SEG_PER_W * D)])
        pltpu.sync_copy(seen_v, seen_hbm.at[pl.ds(s0, SEG_PER_W)])

    return seg_max(x_mod_flat, ptr32)


def kernel(x_main, x_mod, csr_idx):
    del x_main
    n_seg = csr_idx.shape[0] - 1
    ptr32 = jnp.concatenate(
        [
            csr_idx.astype(jnp.int32),
            jnp.full((PTR_PAD - n_seg - 1,), N_EDGES, jnp.int32),
        ]
    )
    pool_pad, seen_pad = _seg_max_sc(x_mod.reshape(-1), ptr32)
    pool = pool_pad.reshape(N_SEG_PAD, D)[:n_seg]
    return (pool, seen_pad[:n_seg].astype(bool))


# speedup vs baseline: 215.0015x; 1.0415x over previous
import functools

import jax
import jax.numpy as jnp
from jax import lax
from jax.experimental import pallas as pl
from jax.experimental.pallas import tpu as pltpu
from jax.experimental.pallas import tpu_sc as plsc

N_NODES = 10000
N_EDGES = 320000
D = 128

NC = 2
NS = 16
NW = NC * NS

SEG_PER_W = 320
N_SEG_PAD = NW * SEG_PER_W
PTR_SLICE = 344
PTR_PAD = (NW - 1) * SEG_PER_W + PTR_SLICE
CH = 320
NLANES = D // 16


def _seg_max_sc(x_mod_flat, ptr32):
    mesh = plsc.VectorSubcoreMesh(
        core_axis_name="c", subcore_axis_name="s", num_cores=NC, num_subcores=NS
    )

    @functools.partial(
        pl.kernel,
        out_type=[
            jax.ShapeDtypeStruct((N_SEG_PAD * D,), jnp.float32),
            jax.ShapeDtypeStruct((N_SEG_PAD,), jnp.int32),
        ],
        mesh=mesh,
        scratch_types=[
            pltpu.VMEM((PTR_SLICE,), jnp.int32),
            pltpu.VMEM((SEG_PER_W * D,), jnp.float32),
            pltpu.VMEM((SEG_PER_W,), jnp.int32),
            pltpu.VMEM((2 * CH * D,), jnp.float32),
            pltpu.SemaphoreType.DMA,
            pltpu.SemaphoreType.DMA,
        ],
    )
    def seg_max(x_mod_hbm, ptr_hbm, pool_hbm, seen_hbm,
                ptr_v, acc_v, seen_v, buf_v, sem0, sem1):
        wid = lax.axis_index("s") * NC + lax.axis_index("c")
        s0 = wid * SEG_PER_W

        pltpu.sync_copy(ptr_hbm.at[pl.ds(s0, PTR_SLICE)], ptr_v)

        def ptr_at(i):
            return ptr_v[pl.ds(i, 16)][0]

        e_lo = ptr_at(0)
        e_hi = ptr_at(SEG_PER_W)

        base = (e_lo // 8) * 8
        nchunks = jnp.where(
            e_hi > e_lo, lax.div(e_hi - base + (CH - 1), CH), 0
        )

        def chunk_start(c):
            st = jnp.minimum(base + c * CH, N_EDGES - CH)
            return pl.multiple_of(st, 8)

        def start_dma(c, b, sem):
            pltpu.make_async_copy(
                x_mod_hbm.at[pl.ds(chunk_start(c) * D, CH * D)],
                buf_v.at[pl.ds(b * CH * D, CH * D)],
                sem,
            ).start()

        def wait_dma(b, sem):
            pltpu.make_async_copy(
                x_mod_hbm.at[pl.ds(0, CH * D)],
                buf_v.at[pl.ds(b * CH * D, CH * D)],
                sem,
            ).wait()

        @pl.when(nchunks > 0)
        def _():
            start_dma(0, 0, sem0)

        @pl.when(nchunks > 1)
        def _():
            start_dma(1, 1, sem1)

        zeros16 = jnp.zeros((16,), jnp.float32)

        def zrow(r, _):
            for j in range(NLANES):
                acc_v[pl.ds(r * D + 16 * j, 16)] = zeros16
            return 0

        lax.fori_loop(0, SEG_PER_W, zrow, 0)

        for k in range(SEG_PER_W // 16):
            a = ptr_v[pl.ds(16 * k, 16)]
            b = ptr_v[pl.ds(16 * k + 1, 16)]
            seen_v[pl.ds(16 * k, 16)] = jnp.where(b > a, 1, 0).astype(jnp.int32)

        minus_inf = jnp.full((16,), -jnp.inf, jnp.float32)

        def seg_body(s, ops):
            res, start = ops
            end = ptr_at(s + 1)

            c_first = lax.div(start - base, CH)
            c_last = jnp.where(
                end > start, lax.div(end - 1 - base, CH), c_first - 1
            )

            def chunk_body(c, cops):
                res, *acc = cops

                def do_cross(_):
                    @pl.when(lax.rem(c, 2) == 0)
                    def _():
                        wait_dma(0, sem0)

                    @pl.when(lax.rem(c, 2) == 1)
                    def _():
                        wait_dma(1, sem1)

                    nc = c + 1

                    @pl.when((c > 0) & (nc < nchunks))
                    def _():
                        @pl.when(lax.rem(nc, 2) == 0)
                        def _():
                            start_dma(nc, 0, sem0)

                        @pl.when(lax.rem(nc, 2) == 1)
                        def _():
                            start_dma(nc, 1, sem1)

                    return c

                res = lax.cond(c > res, do_cross, lambda _: res, 0)

                st_dma = jnp.minimum(base + c * CH, N_EDGES - CH)
                off = (lax.rem(c, 2) * CH - st_dma) * D
                lo = jnp.maximum(start, base + c * CH)
                hi = jnp.minimum(end, base + (c + 1) * CH)

                def row_body(e, acc):
                    bi = off + e * D
                    return tuple(
                        jnp.maximum(acc[j], buf_v[pl.ds(bi + 16 * j, 16)])
                        for j in range(NLANES)
                    )

                acc = lax.fori_loop(lo, hi, row_body, tuple(acc))
                return (res,) + acc

            init = (res,) + tuple([minus_inf] * NLANES)
            res, *acc = lax.fori_loop(c_first, c_last + 1, chunk_body, init)

            @pl.when(end > start)
            def _():
                for j in range(NLANES):
                    acc_v[pl.ds(s * D + 16 * j, 16)] = acc[j]

            return (res, end)

        lax.fori_loop(0, SEG_PER_W, seg_body, (jnp.int32(-1), e_lo))

        pltpu.sync_copy(acc_v, pool_hbm.at[pl.ds(s0 * D, SEG_PER_W * D)])
        pltpu.sync_copy(seen_v, seen_hbm.at[pl.ds(s0, SEG_PER_W)])

    return seg_max(x_mod_flat, ptr32)


def kernel(x_main, x_mod, csr_idx):
    del x_main
    n_seg = csr_idx.shape[0] - 1
    ptr32 = jnp.concatenate(
        [
            csr_idx.astype(jnp.int32),
            jnp.full((PTR_PAD - n_seg - 1,), N_EDGES, jnp.int32),
        ]
    )
    pool_pad, seen_pad = _seg_max_sc(x_mod.reshape(-1), ptr32)
    pool = pool_pad.reshape(N_SEG_PAD, D)[:n_seg]
    return (pool, seen_pad[:n_seg].astype(bool))


# speedup vs baseline: 221.6540x; 1.0309x over previous
import functools

import jax
import jax.numpy as jnp
from jax import lax
from jax.experimental import pallas as pl
from jax.experimental.pallas import tpu as pltpu
from jax.experimental.pallas import tpu_sc as plsc

N_NODES = 10000
N_EDGES = 320000
D = 128

NC = 2
NS = 16
NW = NC * NS

SEG_PER_W = 320
N_SEG_PAD = NW * SEG_PER_W
PTR_SLICE = 344
PTR_PAD = (NW - 1) * SEG_PER_W + PTR_SLICE
CH = 320
NLANES = D // 16
LAST_W = N_NODES - (NW - 1) * SEG_PER_W


def _seg_max_sc(x_mod_flat, ptr32):
    mesh = plsc.VectorSubcoreMesh(
        core_axis_name="c", subcore_axis_name="s", num_cores=NC, num_subcores=NS
    )

    @functools.partial(
        pl.kernel,
        out_type=[
            jax.ShapeDtypeStruct((N_NODES * D,), jnp.float32),
            jax.ShapeDtypeStruct((N_NODES,), jnp.int32),
        ],
        mesh=mesh,
        scratch_types=[
            pltpu.VMEM((PTR_SLICE,), jnp.int32),
            pltpu.VMEM((SEG_PER_W * D,), jnp.float32),
            pltpu.VMEM((SEG_PER_W,), jnp.int32),
            pltpu.VMEM((2 * CH * D,), jnp.float32),
            pltpu.SemaphoreType.DMA,
            pltpu.SemaphoreType.DMA,
        ],
    )
    def seg_max(x_mod_hbm, ptr_hbm, pool_hbm, seen_hbm,
                ptr_v, acc_v, seen_v, buf_v, sem0, sem1):
        wid = lax.axis_index("s") * NC + lax.axis_index("c")
        s0 = wid * SEG_PER_W

        pltpu.sync_copy(ptr_hbm.at[pl.ds(s0, PTR_SLICE)], ptr_v)

        def ptr_at(i):
            return ptr_v[pl.ds(i, 16)][0]

        e_lo = ptr_at(0)
        e_hi = ptr_at(SEG_PER_W)

        base = (e_lo // 8) * 8
        nchunks = jnp.where(
            e_hi > e_lo, lax.div(e_hi - base + (CH - 1), CH), 0
        )

        def chunk_start(c):
            st = jnp.minimum(base + c * CH, N_EDGES - CH)
            return pl.multiple_of(st, 8)

        def start_dma(c, b, sem):
            pltpu.make_async_copy(
                x_mod_hbm.at[pl.ds(chunk_start(c) * D, CH * D)],
                buf_v.at[pl.ds(b * CH * D, CH * D)],
                sem,
            ).start()

        def wait_dma(b, sem):
            pltpu.make_async_copy(
                x_mod_hbm.at[pl.ds(0, CH * D)],
                buf_v.at[pl.ds(b * CH * D, CH * D)],
                sem,
            ).wait()

        @pl.when(nchunks > 0)
        def _():
            start_dma(0, 0, sem0)

        @pl.when(nchunks > 1)
        def _():
            start_dma(1, 1, sem1)

        zeros16 = jnp.zeros((16,), jnp.float32)

        def zrow(r, _):
            for j in range(NLANES):
                acc_v[pl.ds(r * D + 16 * j, 16)] = zeros16
            return 0

        lax.fori_loop(0, SEG_PER_W, zrow, 0)

        for k in range(SEG_PER_W // 16):
            a = ptr_v[pl.ds(16 * k, 16)]
            b = ptr_v[pl.ds(16 * k + 1, 16)]
            seen_v[pl.ds(16 * k, 16)] = jnp.where(b > a, 1, 0).astype(jnp.int32)

        minus_inf = jnp.full((16,), -jnp.inf, jnp.float32)

        def seg_body(s, ops):
            res, start = ops
            end = ptr_at(s + 1)

            c_first = lax.div(start - base, CH)
            c_last = jnp.where(
                end > start, lax.div(end - 1 - base, CH), c_first - 1
            )

            def chunk_body(c, cops):
                res, *acc = cops

                def do_cross(_):
                    @pl.when(lax.rem(c, 2) == 0)
                    def _():
                        wait_dma(0, sem0)

                    @pl.when(lax.rem(c, 2) == 1)
                    def _():
                        wait_dma(1, sem1)

                    nc = c + 1

                    @pl.when((c > 0) & (nc < nchunks))
                    def _():
                        @pl.when(lax.rem(nc, 2) == 0)
                        def _():
                            start_dma(nc, 0, sem0)

                        @pl.when(lax.rem(nc, 2) == 1)
                        def _():
                            start_dma(nc, 1, sem1)

                    return c

                res = lax.cond(c > res, do_cross, lambda _: res, 0)

                st_dma = jnp.minimum(base + c * CH, N_EDGES - CH)
                off = (lax.rem(c, 2) * CH - st_dma) * D
                lo = jnp.maximum(start, base + c * CH)
                hi = jnp.minimum(end, base + (c + 1) * CH)

                def row_body(e, acc):
                    bi = off + e * D
                    return tuple(
                        jnp.maximum(acc[j], buf_v[pl.ds(bi + 16 * j, 16)])
                        for j in range(NLANES)
                    )

                acc = lax.fori_loop(lo, hi, row_body, tuple(acc))
                return (res,) + acc

            init = (res,) + tuple([minus_inf] * NLANES)
            res, *acc = lax.fori_loop(c_first, c_last + 1, chunk_body, init)

            @pl.when(end > start)
            def _():
                for j in range(NLANES):
                    acc_v[pl.ds(s * D + 16 * j, 16)] = acc[j]

            return (res, end)

        lax.fori_loop(0, SEG_PER_W, seg_body, (jnp.int32(-1), e_lo))

        @pl.when(wid < NW - 1)
        def _():
            pltpu.sync_copy(acc_v, pool_hbm.at[pl.ds(s0 * D, SEG_PER_W * D)])
            pltpu.sync_copy(seen_v, seen_hbm.at[pl.ds(s0, SEG_PER_W)])

        @pl.when(wid == NW - 1)
        def _():
            pltpu.sync_copy(
                acc_v.at[pl.ds(0, LAST_W * D)],
                pool_hbm.at[pl.ds(s0 * D, LAST_W * D)],
            )
            pltpu.sync_copy(
                seen_v.at[pl.ds(0, LAST_W)], seen_hbm.at[pl.ds(s0, LAST_W)]
            )

    return seg_max(x_mod_flat, ptr32)


def kernel(x_main, x_mod, csr_idx):
    del x_main
    n_seg = csr_idx.shape[0] - 1
    ptr32 = jnp.concatenate(
        [
            csr_idx.astype(jnp.int32),
            jnp.full((PTR_PAD - n_seg - 1,), N_EDGES, jnp.int32),
        ]
    )
    pool_flat, seen_i32 = _seg_max_sc(x_mod.reshape(-1), ptr32)
    return (pool_flat.reshape(n_seg, D), seen_i32.astype(bool))


# speedup vs baseline: 240.4251x; 1.0847x over previous
import functools

import jax
import jax.numpy as jnp
from jax import lax
from jax.experimental import pallas as pl
from jax.experimental.pallas import tpu as pltpu
from jax.experimental.pallas import tpu_sc as plsc

N_NODES = 10000
N_EDGES = 320000
D = 128

NC = 2
NS = 16
NW = NC * NS

SEG_PER_W = 320
N_SEG_PAD = NW * SEG_PER_W
PTR_SLICE = 344
PTR_PAD = (NW - 1) * SEG_PER_W + PTR_SLICE
CH = 224
NBUF = 3
NLANES = D // 16
LAST_W = N_NODES - (NW - 1) * SEG_PER_W


def _seg_max_sc(x_mod_flat, ptr32):
    mesh = plsc.VectorSubcoreMesh(
        core_axis_name="c", subcore_axis_name="s", num_cores=NC, num_subcores=NS
    )

    @functools.partial(
        pl.kernel,
        out_type=[
            jax.ShapeDtypeStruct((N_NODES * D,), jnp.float32),
            jax.ShapeDtypeStruct((N_NODES,), jnp.int32),
        ],
        mesh=mesh,
        scratch_types=[
            pltpu.VMEM((PTR_SLICE,), jnp.int32),
            pltpu.VMEM((SEG_PER_W * D,), jnp.float32),
            pltpu.VMEM((SEG_PER_W,), jnp.int32),
            pltpu.VMEM((NBUF * CH * D,), jnp.float32),
            pltpu.SemaphoreType.DMA,
            pltpu.SemaphoreType.DMA,
            pltpu.SemaphoreType.DMA,
        ],
    )
    def seg_max(x_mod_hbm, ptr_hbm, pool_hbm, seen_hbm,
                ptr_v, acc_v, seen_v, buf_v, sem0, sem1, sem2):
        sems = [sem0, sem1, sem2]
        wid = lax.axis_index("s") * NC + lax.axis_index("c")
        s0 = wid * SEG_PER_W

        pltpu.sync_copy(ptr_hbm.at[pl.ds(s0, PTR_SLICE)], ptr_v)

        def ptr_at(i):
            return ptr_v[pl.ds(i, 16)][0]

        e_lo = ptr_at(0)
        e_hi = ptr_at(SEG_PER_W)

        base = (e_lo // 8) * 8
        nchunks = jnp.where(
            e_hi > e_lo, lax.div(e_hi - base + (CH - 1), CH), 0
        )

        def chunk_start(c):
            st = jnp.minimum(base + c * CH, N_EDGES - CH)
            return pl.multiple_of(st, 8)

        def start_dma(c, b, sem):
            pltpu.make_async_copy(
                x_mod_hbm.at[pl.ds(chunk_start(c) * D, CH * D)],
                buf_v.at[pl.ds(b * CH * D, CH * D)],
                sem,
            ).start()

        def wait_dma(b, sem):
            pltpu.make_async_copy(
                x_mod_hbm.at[pl.ds(0, CH * D)],
                buf_v.at[pl.ds(b * CH * D, CH * D)],
                sem,
            ).wait()

        @pl.when(nchunks > 0)
        def _():
            start_dma(0, 0, sem0)

        @pl.when(nchunks > 1)
        def _():
            start_dma(1, 1, sem1)

        zeros16 = jnp.zeros((16,), jnp.float32)

        def zrow(r, _):
            for j in range(NLANES):
                acc_v[pl.ds(r * D + 16 * j, 16)] = zeros16
            return 0

        lax.fori_loop(0, SEG_PER_W, zrow, 0)

        for k in range(SEG_PER_W // 16):
            a = ptr_v[pl.ds(16 * k, 16)]
            b = ptr_v[pl.ds(16 * k + 1, 16)]
            seen_v[pl.ds(16 * k, 16)] = jnp.where(b > a, 1, 0).astype(jnp.int32)

        minus_inf = jnp.full((16,), -jnp.inf, jnp.float32)

        def seg_body(s, ops):
            res, start = ops
            end = ptr_at(s + 1)

            c_first = lax.div(start - base, CH)
            c_last = jnp.where(
                end > start, lax.div(end - 1 - base, CH), c_first - 1
            )

            def chunk_body(c, cops):
                res, *acc = cops

                def do_cross(_):
                    cm = lax.rem(c, NBUF)
                    for b in range(NBUF):
                        @pl.when(cm == b)
                        def _(b=b):
                            wait_dma(b, sems[b])

                    nc = c + 2

                    @pl.when(nc < nchunks)
                    def _():
                        ncm = lax.rem(nc, NBUF)
                        for b in range(NBUF):
                            @pl.when(ncm == b)
                            def _(b=b):
                                start_dma(nc, b, sems[b])

                    return c

                res = lax.cond(c > res, do_cross, lambda _: res, 0)

                st_dma = jnp.minimum(base + c * CH, N_EDGES - CH)
                off = (lax.rem(c, NBUF) * CH - st_dma) * D
                lo = jnp.maximum(start, base + c * CH)
                hi = jnp.minimum(end, base + (c + 1) * CH)

                def row_body(e, acc):
                    bi = off + e * D
                    return tuple(
                        jnp.maximum(acc[j], buf_v[pl.ds(bi + 16 * j, 16)])
                        for j in range(NLANES)
                    )

                acc = lax.fori_loop(lo, hi, row_body, tuple(acc))
                return (res,) + acc

            init = (res,) + tuple([minus_inf] * NLANES)
            res, *acc = lax.fori_loop(c_first, c_last + 1, chunk_body, init)

            @pl.when(end > start)
            def _():
                for j in range(NLANES):
                    acc_v[pl.ds(s * D + 16 * j, 16)] = acc[j]

            return (res, end)

        lax.fori_loop(0, SEG_PER_W, seg_body, (jnp.int32(-1), e_lo))

        @pl.when(wid < NW - 1)
        def _():
            pltpu.sync_copy(acc_v, pool_hbm.at[pl.ds(s0 * D, SEG_PER_W * D)])
            pltpu.sync_copy(seen_v, seen_hbm.at[pl.ds(s0, SEG_PER_W)])

        @pl.when(wid == NW - 1)
        def _():
            pltpu.sync_copy(
                acc_v.at[pl.ds(0, LAST_W * D)],
                pool_hbm.at[pl.ds(s0 * D, LAST_W * D)],
            )
            pltpu.sync_copy(
                seen_v.at[pl.ds(0, LAST_W)], seen_hbm.at[pl.ds(s0, LAST_W)]
            )

    return seg_max(x_mod_flat, ptr32)


def kernel(x_main, x_mod, csr_idx):
    del x_main
    n_seg = csr_idx.shape[0] - 1
    ptr32 = jnp.concatenate(
        [
            csr_idx.astype(jnp.int32),
            jnp.full((PTR_PAD - n_seg - 1,), N_EDGES, jnp.int32),
        ]
    )
    pool_flat, seen_i32 = _seg_max_sc(x_mod.reshape(-1), ptr32)
    return (pool_flat.reshape(n_seg, D), seen_i32.astype(bool))
